# Initial kernel scaffold; baseline (speedup 1.0000x reference)
import jax
import jax.numpy as jnp
from jax.experimental import pallas as pl


def kernel(x1, edge_index1, batch1, x2, edge_index2, batch2, cell, params):
    raise NotImplementedError("write your pallas kernel here")


# speedup vs baseline: 1.0011x; 1.0011x over previous
import functools

import jax
import jax.numpy as jnp
from jax.experimental import pallas as pl
from jax.experimental.pallas import tpu as pltpu

NUM_G = 128
N_NODES = 10000


def _lrelu(x, s=0.01):
    return jnp.where(x >= 0, x, s * x)


def _tail_body(x1_ref, b1_ref, x2_ref, b2_ref, cell_ref, *prefs, out_ref):
    names = _TAIL_PARAM_NAMES
    p = {n: r[...] for n, r in zip(names, prefs)}

    def drug_tail(x, batch, tag):
        z = x @ p[f"{tag}att_w"] + p[f"{tag}att_b"]
        z = z - jnp.max(z)
        ez = jnp.exp(z)
        s = ez / jnp.sum(ez)
        gid = jax.lax.broadcasted_iota(jnp.int32, (1, NUM_G), 1)
        oh = (batch == gid).astype(jnp.float32)
        sx = s * x
        g = jax.lax.dot_general(oh, sx, (((0,), (0,)), ((), ())),
                                preferred_element_type=jnp.float32)
        h = _lrelu((g @ p[f"{tag}fc1_w"] + p[f"{tag}fc1_b"]) * p[f"{tag}bn1_g"]
                   + p[f"{tag}bn1_b"])
        return h @ p[f"{tag}fc2_w"] + p[f"{tag}fc2_b"]

    h1 = drug_tail(x1_ref[...], b1_ref[...], "d1.")
    h2 = drug_tail(x2_ref[...], b2_ref[...], "d2.")

    cell = cell_ref[...]
    v = cell / (jnp.sqrt(jnp.sum(cell * cell, axis=1, keepdims=True)) + 1e-12)
    h = _lrelu((v @ p["c.w1"] + p["c.b1"]) * p["c.g1"] + p["c.be1"])
    h = _lrelu((h @ p["c.w2"] + p["c.b2"]) * p["c.g2"] + p["c.be2"])
    c = h @ p["c.w3"] + p["c.b3"]

    xc = jnp.concatenate([h1, h2, c], axis=1)
    for i in range(2):
        gate = jax.nn.sigmoid(xc @ p[f"m.gw{i}"] + p[f"m.gb{i}"])
        nl = _lrelu(xc @ p[f"m.nw{i}"] + p[f"m.nb{i}"])
        lin = xc @ p[f"m.lw{i}"] + p[f"m.lb{i}"]
        xc = gate * nl + (1.0 - gate) * lin
    h = _lrelu(xc @ p["s.w1"] + p["s.b1"])
    h = _lrelu(h @ p["s.w2"] + p["s.b2"])
    out_ref[...] = h @ p["s.w3"] + p["s.b3"]


_TAIL_PARAM_NAMES = None


def _run_tail(x1, batch1, x2, batch2, cell, params):
    global _TAIL_PARAM_NAMES
    tail = {}
    for tag, d in (("d1.", params["d1"]), ("d2.", params["d2"])):
        for k in ("att_w", "att_b", "fc1_w", "fc1_b", "bn1_g", "bn1_b",
                  "fc2_w", "fc2_b"):
            tail[tag + k] = d[k]
    for k, v in params["cell"].items():
        tail["c." + k] = v
    for k, v in params["mfic"].items():
        tail["m." + k] = v
    for k, v in params["syn"].items():
        tail["s." + k] = v
    _TAIL_PARAM_NAMES = tuple(tail.keys())
    pvals = [tail[n] for n in _TAIL_PARAM_NAMES]

    body = lambda *refs: _tail_body(*refs[:-1], out_ref=refs[-1])
    return pl.pallas_call(
        body,
        out_shape=jax.ShapeDtypeStruct((NUM_G, 2), jnp.float32),
    )(x1, batch1.reshape(N_NODES, 1), x2, batch2.reshape(N_NODES, 1),
      cell, *pvals)


def _gat(x, src, dst, W, a_s, a_d, b):
    N = x.shape[0]
    h = x @ W
    e = _lrelu((h @ a_s)[src] + (h @ a_d)[dst], 0.2)
    m = jax.ops.segment_max(e, dst, num_segments=N)
    m = jnp.where(jnp.isfinite(m), m, 0.0)
    ex = jnp.exp(e - m[dst])
    den = jax.ops.segment_sum(ex, dst, num_segments=N)
    alpha = ex / (den[dst] + 1e-16)
    return jax.ops.segment_sum(h[src] * alpha[:, None], dst, num_segments=N) + b


def _drug_graph(x, edge_index, p):
    N = x.shape[0]
    loop = jnp.arange(N, dtype=edge_index.dtype)
    src = jnp.concatenate([edge_index[0], loop])
    dst = jnp.concatenate([edge_index[1], loop])
    for i in range(3):
        x = _lrelu(_gat(x, src, dst, p[f"W{i}"], p[f"as{i}"], p[f"ad{i}"],
                        p[f"b{i}"]))
    return x


@jax.jit
def kernel(x1, edge_index1, batch1, x2, edge_index2, batch2, cell, params):
    x1f = _drug_graph(x1, edge_index1, params["d1"])
    x2f = _drug_graph(x2, edge_index2, params["d2"])
    return _run_tail(x1f, batch1, x2f, batch2, cell, params)


# speedup vs baseline: 24.1751x; 24.1496x over previous
import functools

import jax
import jax.numpy as jnp
from jax import lax
from jax.experimental import pallas as pl
from jax.experimental.pallas import tpu as pltpu
from jax.experimental.pallas import tpu_sc as plsc

NUM_G = 128
N = 10000
NP = 10112
GARBAGE = N
E_RAW = 640000
E_TOT = E_RAW + N
E_PAD = 655360
EPT = E_PAD // 16
CHUNK = 128
NCHUNK = EPT // CHUNK
RPT = NP // 16

_F_IN = (78, 32, 64)
_F_OUT = (32, 64, 128)


def _lrelu(x, s=0.01):
    return jnp.where(x >= 0, x, s * x)


def _prep0_body(x1, x2, W1, as1, ad1, W2, as2, ad2, h_o, es_o, ed_o, m_o):
    for d, (x_r, W_r, as_r, ad_r) in enumerate(
            ((x1, W1, as1, ad1), (x2, W2, as2, ad2))):
        h = x_r[...] @ W_r[...]
        es = h @ as_r[...]
        ed = h @ ad_r[...]
        M = jnp.maximum(0.0, jnp.max(es) + jnp.max(ed))
        z1 = jnp.zeros((NP - N, 1), jnp.float32)
        hp = jnp.concatenate(
            [h, jnp.zeros((N, 128 - h.shape[1]), jnp.float32)], axis=1)
        hp = jnp.concatenate(
            [hp, jnp.zeros((NP - N, 128), jnp.float32)], axis=0)
        h_o[pl.ds(d * NP, NP)] = hp
        es_o[pl.ds(d * NP, NP)] = jnp.concatenate([es, z1], axis=0)
        ed_o[pl.ds(d * NP, NP)] = jnp.concatenate([ed, z1], axis=0)
        m_o[pl.ds(d, 1)] = jnp.full((1, 16), M, jnp.float32)


def _prepmid_body(acc, den, b1, W1, as1, ad1, b2, W2, as2, ad2,
                  h_o, es_o, ed_o, m_o):
    row = lax.broadcasted_iota(jnp.int32, (NP, 1), 0)
    Fp = b1.shape[1]
    for d, (b_r, W_r, as_r, ad_r) in enumerate(
            ((b1, W1, as1, ad1), (b2, W2, as2, ad2))):
        g = acc[d][:, :Fp] / (den[d] + 1e-16) + b_r[...]
        x = jnp.where(row < N, _lrelu(g), 0.0)
        h = x @ W_r[...]
        es = h @ as_r[...]
        ed = h @ ad_r[...]
        M = jnp.maximum(0.0, jnp.max(es) + jnp.max(ed))
        if h.shape[1] < 128:
            h = jnp.concatenate(
                [h, jnp.zeros((NP, 128 - h.shape[1]), jnp.float32)], axis=1)
        h_o[pl.ds(d * NP, NP)] = h
        es_o[pl.ds(d * NP, NP)] = es
        ed_o[pl.ds(d * NP, NP)] = ed
        m_o[pl.ds(d, 1)] = jnp.full((1, 16), M, jnp.float32)


def _sc_body(src_f, dst_f, es_f, ed_f, m_f, h_f,
             acc_o, den_o,
             es_t, ed_t, m_b, src_b, dst_b, srcg_b, ex_b, rows,
             acc_s, den_s, gsem):
    cid = lax.axis_index("c")
    sid = lax.axis_index("s")
    ebase = cid * E_PAD + sid * EPT
    r0 = sid * RPT

    pltpu.sync_copy(es_f.at[pl.ds(cid * NP, NP)], es_t)
    pltpu.sync_copy(ed_f.at[pl.ds(cid * NP, NP)], ed_t)
    pltpu.sync_copy(m_f.at[pl.ds(cid * 16, 16)], m_b)

    zv = jnp.zeros((16,), jnp.float32)

    def zrow(i, c):
        for j in range(128 // 16):
            rows[(i, pl.ds(j * 16, 16))] = zv
        return c

    lax.fori_loop(0, CHUNK, zrow, 0)
    for j in range(CHUNK // 16):
        ex_b[pl.ds(j * 16, 16)] = zv
    for off in range(0, RPT, CHUNK):
        sz = min(CHUNK, RPT - off)
        pltpu.sync_copy(rows.at[pl.ds(0, sz)], acc_s.at[pl.ds(r0 + off, sz)])
        pltpu.sync_copy(ex_b.at[pl.ds(0, sz)], den_s.at[pl.ds(r0 + off, sz)])
    plsc.subcore_barrier()

    mv = m_b[...]
    goff = cid * NP

    def chunk(k, _):
        cb = ebase + k * CHUNK
        pltpu.sync_copy(src_f.at[pl.ds(cb, CHUNK)], src_b)
        pltpu.sync_copy(dst_f.at[pl.ds(cb, CHUNK)], dst_b)
        for j in range(CHUNK // 16):
            sl = pl.ds(j * 16, 16)
            sv = src_b[sl]
            dv = dst_b[sl]
            e = plsc.load_gather(es_t, [sv]) + plsc.load_gather(ed_t, [dv])
            e = jnp.where(e >= 0, e, 0.2 * e) - mv
            ex_b[sl] = jnp.exp(e)
            srcg_b[sl] = sv + goff
        pltpu.sync_copy(ex_b, den_s.at[dst_b], add=True)
        pltpu.async_copy(h_f.at[srcg_b], rows, gsem).wait()

        def scale(i, carry):
            s = plsc.load_gather(ex_b, [jnp.full((16,), 0, jnp.int32) + i])
            for j in range(128 // 16):
                c = (i, pl.ds(j * 16, 16))
                rows[c] = rows[c] * s
            return carry

        lax.fori_loop(0, CHUNK, scale, 0)
        pltpu.sync_copy(rows, acc_s.at[dst_b], add=True)
        return _

    lax.fori_loop(0, NCHUNK, chunk, 0)
    plsc.subcore_barrier()

    for off in range(0, RPT, CHUNK):
        sz = min(CHUNK, RPT - off)
        pltpu.sync_copy(acc_s.at[pl.ds(r0 + off, sz)], rows.at[pl.ds(0, sz)])
        pltpu.sync_copy(rows.at[pl.ds(0, sz)],
                        acc_o.at[cid, pl.ds(r0 + off, sz)])
        pltpu.sync_copy(den_s.at[pl.ds(r0 + off, sz)], ex_b.at[pl.ds(0, sz)])
        pltpu.sync_copy(ex_b.at[pl.ds(0, sz)],
                        den_o.at[pl.ds(cid * NP + r0 + off, sz)])


def _sc_layer():
    mesh = plsc.VectorSubcoreMesh(core_axis_name="c", subcore_axis_name="s")
    return pl.kernel(
        _sc_body,
        out_type=(jax.ShapeDtypeStruct((2, NP, 128), jnp.float32),
                  jax.ShapeDtypeStruct((2 * NP,), jnp.float32)),
        mesh=mesh,
        compiler_params=pltpu.CompilerParams(needs_layout_passes=False),
        scratch_types=[
            pltpu.VMEM((NP,), jnp.float32),
            pltpu.VMEM((NP,), jnp.float32),
            pltpu.VMEM((16,), jnp.float32),
            pltpu.VMEM((CHUNK,), jnp.int32),
            pltpu.VMEM((CHUNK,), jnp.int32),
            pltpu.VMEM((CHUNK,), jnp.int32),
            pltpu.VMEM((CHUNK,), jnp.float32),
            pltpu.VMEM((CHUNK, 128), jnp.float32),
            pltpu.VMEM_SHARED((NP, 128), jnp.float32),
            pltpu.VMEM_SHARED((NP,), jnp.float32),
            pltpu.SemaphoreType.DMA,
        ],
    )


def _tail_body(acc, den, b1_ref, b2_ref, cb1_ref, cb2_ref, cell_ref, *rest):
    prefs = rest[:-1]
    out_ref = rest[-1]
    names = _TAIL_PARAM_NAMES
    p = {n: r[...] for n, r in zip(names, prefs)}
    row = lax.broadcasted_iota(jnp.int32, (NP, 1), 0)

    def drug_tail(d, blast, batch, tag):
        g = acc[d] / (den[d] + 1e-16) + blast[...]
        x = jnp.where(row < N, _lrelu(g), 0.0)
        z = x @ p[f"{tag}att_w"] + p[f"{tag}att_b"]
        z = jnp.where(row < N, z, -1e30)
        z = z - jnp.max(z)
        ez = jnp.exp(z)
        s = ez / jnp.sum(ez)
        gid = lax.broadcasted_iota(jnp.int32, (1, NUM_G), 1)
        oh = (batch[...] == gid).astype(jnp.float32)
        gp = lax.dot_general(oh, s * x, (((0,), (0,)), ((), ())),
                             preferred_element_type=jnp.float32)
        h = _lrelu((gp @ p[f"{tag}fc1_w"] + p[f"{tag}fc1_b"]) * p[f"{tag}bn1_g"]
                   + p[f"{tag}bn1_b"])
        return h @ p[f"{tag}fc2_w"] + p[f"{tag}fc2_b"]

    h1 = drug_tail(0, b1_ref, cb1_ref, "d1.")
    h2 = drug_tail(1, b2_ref, cb2_ref, "d2.")

    cell = cell_ref[...]
    v = cell / (jnp.sqrt(jnp.sum(cell * cell, axis=1, keepdims=True)) + 1e-12)
    h = _lrelu((v @ p["c.w1"] + p["c.b1"]) * p["c.g1"] + p["c.be1"])
    h = _lrelu((h @ p["c.w2"] + p["c.b2"]) * p["c.g2"] + p["c.be2"])
    c = h @ p["c.w3"] + p["c.b3"]

    xc = jnp.concatenate([h1, h2, c], axis=1)
    for i in range(2):
        gate = jax.nn.sigmoid(xc @ p[f"m.gw{i}"] + p[f"m.gb{i}"])
        nl = _lrelu(xc @ p[f"m.nw{i}"] + p[f"m.nb{i}"])
        lin = xc @ p[f"m.lw{i}"] + p[f"m.lb{i}"]
        xc = gate * nl + (1.0 - gate) * lin
    h = _lrelu(xc @ p["s.w1"] + p["s.b1"])
    h = _lrelu(h @ p["s.w2"] + p["s.b2"])
    out_ref[...] = h @ p["s.w3"] + p["s.b3"]


_TAIL_PARAM_NAMES = None


def _edge_arrays(edge_index):
    loop = jnp.arange(N, dtype=jnp.int32)
    padn = E_PAD - E_TOT
    src = jnp.concatenate([edge_index[0], loop,
                           jnp.zeros((padn,), jnp.int32)])
    dst = jnp.concatenate([edge_index[1], loop,
                           jnp.full((padn,), GARBAGE, jnp.int32)])
    return src, dst


@jax.jit
def kernel(x1, edge_index1, batch1, x2, edge_index2, batch2, cell, params):
    global _TAIL_PARAM_NAMES
    d1, d2 = params["d1"], params["d2"]

    src1, dst1 = _edge_arrays(edge_index1)
    src2, dst2 = _edge_arrays(edge_index2)
    src_f = jnp.concatenate([src1, src2])
    dst_f = jnp.concatenate([dst1, dst2])

    h, es, ed, m = pl.pallas_call(
        _prep0_body,
        out_shape=(jax.ShapeDtypeStruct((2 * NP, 128), jnp.float32),
                   jax.ShapeDtypeStruct((2 * NP, 1), jnp.float32),
                   jax.ShapeDtypeStruct((2 * NP, 1), jnp.float32),
                   jax.ShapeDtypeStruct((2, 16), jnp.float32)),
    )(x1, x2,
      d1["W0"], d1["as0"].reshape(-1, 1), d1["ad0"].reshape(-1, 1),
      d2["W0"], d2["as0"].reshape(-1, 1), d2["ad0"].reshape(-1, 1))

    for i in range(3):
        acc, den = _sc_layer()(
            src_f, dst_f, es.reshape(-1), ed.reshape(-1), m.reshape(-1), h)
        den = den.reshape(2, NP, 1)
        if i < 2:
            h, es, ed, m = pl.pallas_call(
                _prepmid_body,
                out_shape=(jax.ShapeDtypeStruct((2 * NP, 128), jnp.float32),
                           jax.ShapeDtypeStruct((2 * NP, 1), jnp.float32),
                           jax.ShapeDtypeStruct((2 * NP, 1), jnp.float32),
                           jax.ShapeDtypeStruct((2, 16), jnp.float32)),
            )(acc, den,
              d1[f"b{i}"].reshape(1, -1), d1[f"W{i+1}"],
              d1[f"as{i+1}"].reshape(-1, 1), d1[f"ad{i+1}"].reshape(-1, 1),
              d2[f"b{i}"].reshape(1, -1), d2[f"W{i+1}"],
              d2[f"as{i+1}"].reshape(-1, 1), d2[f"ad{i+1}"].reshape(-1, 1))

    tail = {}
    for tag, d in (("d1.", d1), ("d2.", d2)):
        for k in ("att_w", "att_b", "fc1_w", "fc1_b", "bn1_g", "bn1_b",
                  "fc2_w", "fc2_b"):
            tail[tag + k] = d[k]
    for k, v in params["cell"].items():
        tail["c." + k] = v
    for k, v in params["mfic"].items():
        tail["m." + k] = v
    for k, v in params["syn"].items():
        tail["s." + k] = v
    _TAIL_PARAM_NAMES = tuple(tail.keys())
    pvals = [tail[n] for n in _TAIL_PARAM_NAMES]

    bp1 = jnp.pad(batch1, (0, NP - N)).reshape(NP, 1)
    bp2 = jnp.pad(batch2, (0, NP - N)).reshape(NP, 1)

    return pl.pallas_call(
        _tail_body,
        out_shape=jax.ShapeDtypeStruct((NUM_G, 2), jnp.float32),
    )(acc, den, d1["b2"], d2["b2"], bp1, bp2, cell, *pvals)


# speedup vs baseline: 35.5308x; 1.4697x over previous
import functools

import jax
import jax.numpy as jnp
from jax import lax
from jax.experimental import pallas as pl
from jax.experimental.pallas import tpu as pltpu
from jax.experimental.pallas import tpu_sc as plsc

NUM_G = 128
N = 10000
NP = 10112
GARBAGE = N
E_RAW = 640000
E_TOT = E_RAW + N
E_PAD = 655360
EPT = E_PAD // 16
CHUNK = 128
NCHUNK = EPT // CHUNK
RPT = NP // 16

_F_IN = (78, 32, 64)
_F_OUT = (32, 64, 128)


def _lrelu(x, s=0.01):
    return jnp.where(x >= 0, x, s * x)


def _prep0_body(x1, x2, W1, as1, ad1, W2, as2, ad2, h_o, es_o, ed_o, m_o):
    for d, (x_r, W_r, as_r, ad_r) in enumerate(
            ((x1, W1, as1, ad1), (x2, W2, as2, ad2))):
        h = x_r[...] @ W_r[...]
        es = h @ as_r[...]
        ed = h @ ad_r[...]
        M = jnp.maximum(0.0, jnp.max(es) + jnp.max(ed))
        z1 = jnp.zeros((NP - N, 1), jnp.float32)
        hp = jnp.concatenate(
            [h, jnp.zeros((N, 128 - h.shape[1]), jnp.float32)], axis=1)
        hp = jnp.concatenate(
            [hp, jnp.zeros((NP - N, 128), jnp.float32)], axis=0)
        h_o[pl.ds(d * NP, NP)] = hp
        es_o[pl.ds(d * NP, NP)] = jnp.concatenate([es, z1], axis=0)
        ed_o[pl.ds(d * NP, NP)] = jnp.concatenate([ed, z1], axis=0)
        m_o[pl.ds(d, 1)] = jnp.full((1, 16), M, jnp.float32)


def _prepmid_body(acc, den, b1, W1, as1, ad1, b2, W2, as2, ad2,
                  h_o, es_o, ed_o, m_o):
    row = lax.broadcasted_iota(jnp.int32, (NP, 1), 0)
    Fp = b1.shape[1]
    for d, (b_r, W_r, as_r, ad_r) in enumerate(
            ((b1, W1, as1, ad1), (b2, W2, as2, ad2))):
        g = acc[d][:, :Fp] / (den[d] + 1e-16) + b_r[...]
        x = jnp.where(row < N, _lrelu(g), 0.0)
        h = x @ W_r[...]
        es = h @ as_r[...]
        ed = h @ ad_r[...]
        M = jnp.maximum(0.0, jnp.max(es) + jnp.max(ed))
        if h.shape[1] < 128:
            h = jnp.concatenate(
                [h, jnp.zeros((NP, 128 - h.shape[1]), jnp.float32)], axis=1)
        h_o[pl.ds(d * NP, NP)] = h
        es_o[pl.ds(d * NP, NP)] = es
        ed_o[pl.ds(d * NP, NP)] = ed
        m_o[pl.ds(d, 1)] = jnp.full((1, 16), M, jnp.float32)


R = 2


def _sc_body(src_f, dst_f, es_f, ed_f, m_f, h_f,
             acc_o, den_o,
             m_b, src_b, dst_b, srcg_b, ex_b, es_c, ed_c, rows,
             acc_s, den_s, es_sp, ed_sp,
             lsem1, lsem2, esem, edsem, dsem, gsem, ssem):
    cid = lax.axis_index("c")
    sid = lax.axis_index("s")
    ebase = cid * E_PAD + sid * EPT
    r0 = sid * RPT

    pltpu.sync_copy(m_f.at[pl.ds(cid * 16, 16)], m_b)

    zv = jnp.zeros((16,), jnp.float32)

    def zrow(i, c):
        for j in range(128 // 16):
            rows[(0, i, pl.ds(j * 16, 16))] = zv
        return c

    lax.fori_loop(0, CHUNK, zrow, 0)
    for j in range(CHUNK // 16):
        ex_b[(0, pl.ds(j * 16, 16))] = zv
    for off in range(0, RPT, CHUNK):
        sz = min(CHUNK, RPT - off)
        pltpu.sync_copy(rows.at[0, pl.ds(0, sz)],
                        acc_s.at[pl.ds(r0 + off, sz)])
        pltpu.sync_copy(ex_b.at[0, pl.ds(0, sz)],
                        den_s.at[pl.ds(r0 + off, sz)])
        pltpu.sync_copy(es_f.at[pl.ds(cid * NP + r0 + off, sz)],
                        es_c.at[0, pl.ds(0, sz)])
        pltpu.sync_copy(es_c.at[0, pl.ds(0, sz)],
                        es_sp.at[pl.ds(r0 + off, sz)])
        pltpu.sync_copy(ed_f.at[pl.ds(cid * NP + r0 + off, sz)],
                        ed_c.at[0, pl.ds(0, sz)])
        pltpu.sync_copy(ed_c.at[0, pl.ds(0, sz)],
                        ed_sp.at[pl.ds(r0 + off, sz)])
    plsc.subcore_barrier()

    mv = m_b[...]
    goff = cid * NP
    z16 = jnp.full((16,), 0, jnp.int32)

    def load(k, slot):
        cb = ebase + k * CHUNK
        pltpu.async_copy(src_f.at[pl.ds(cb, CHUNK)], src_b.at[slot],
                         lsem1.at[slot])
        pltpu.async_copy(dst_f.at[pl.ds(cb, CHUNK)], dst_b.at[slot],
                         lsem2.at[slot])

    def afterload(slot):
        pltpu.make_async_copy(src_f.at[pl.ds(0, CHUNK)], src_b.at[slot],
                              lsem1.at[slot]).wait()
        pltpu.make_async_copy(dst_f.at[pl.ds(0, CHUNK)], dst_b.at[slot],
                              lsem2.at[slot]).wait()
        pltpu.async_copy(es_sp.at[src_b.at[slot]], es_c.at[slot],
                         esem.at[slot])
        pltpu.async_copy(ed_sp.at[dst_b.at[slot]], ed_c.at[slot],
                         edsem.at[slot])

    def compute(slot):
        pltpu.make_async_copy(es_sp.at[src_b.at[slot]], es_c.at[slot],
                              esem.at[slot]).wait()
        pltpu.make_async_copy(ed_sp.at[dst_b.at[slot]], ed_c.at[slot],
                              edsem.at[slot]).wait()
        for j in range(CHUNK // 16):
            sl = (slot, pl.ds(j * 16, 16))
            e = es_c[sl] + ed_c[sl]
            e = jnp.where(e >= 0, e, 0.2 * e) - mv
            ex_b[sl] = jnp.exp(e)
            srcg_b[sl] = src_b[sl] + goff
        pltpu.async_copy(ex_b.at[slot], den_s.at[dst_b.at[slot]],
                         dsem.at[slot], add=True)
        pltpu.async_copy(h_f.at[srcg_b.at[slot]], rows.at[slot],
                         gsem.at[slot])

    def scale_scatter(slot):
        pltpu.make_async_copy(h_f.at[srcg_b.at[slot]], rows.at[slot],
                              gsem.at[slot]).wait()

        def scale(i, carry):
            s = plsc.load_gather(ex_b, [z16 + slot, z16 + i])
            for j in range(128 // 16):
                c = (slot, i, pl.ds(j * 16, 16))
                rows[c] = rows[c] * s
            return carry

        lax.fori_loop(0, CHUNK, scale, 0)
        pltpu.async_copy(rows.at[slot], acc_s.at[dst_b.at[slot]],
                         ssem.at[slot], add=True)

    def drain(slot):
        pltpu.make_async_copy(rows.at[slot], acc_s.at[dst_b.at[slot]],
                              ssem.at[slot]).wait()
        pltpu.make_async_copy(ex_b.at[slot], den_s.at[dst_b.at[slot]],
                              dsem.at[slot]).wait()

    load(0, 0)
    afterload(0)
    compute(0)
    load(1, 1)
    afterload(1)

    def body(k, _):
        slot = lax.rem(k, R)
        oslot = 1 - slot
        compute(slot)
        scale_scatter(oslot)
        drain(oslot)
        load(k + 1, oslot)
        afterload(oslot)
        return _

    lax.fori_loop(1, NCHUNK, body, 0)

    scale_scatter(lax.rem(NCHUNK - 1, R))
    drain(lax.rem(NCHUNK - 1, R))
    sl_x = lax.rem(NCHUNK, R)
    pltpu.make_async_copy(es_sp.at[src_b.at[sl_x]], es_c.at[sl_x],
                          esem.at[sl_x]).wait()
    pltpu.make_async_copy(ed_sp.at[dst_b.at[sl_x]], ed_c.at[sl_x],
                          edsem.at[sl_x]).wait()
    plsc.subcore_barrier()

    for off in range(0, RPT, CHUNK):
        sz = min(CHUNK, RPT - off)
        pltpu.sync_copy(acc_s.at[pl.ds(r0 + off, sz)],
                        rows.at[0, pl.ds(0, sz)])
        pltpu.sync_copy(rows.at[0, pl.ds(0, sz)],
                        acc_o.at[cid, pl.ds(r0 + off, sz)])
        pltpu.sync_copy(den_s.at[pl.ds(r0 + off, sz)], ex_b.at[0, pl.ds(0, sz)])
        pltpu.sync_copy(ex_b.at[0, pl.ds(0, sz)],
                        den_o.at[pl.ds(cid * NP + r0 + off, sz)])


def _sc_layer():
    mesh = plsc.VectorSubcoreMesh(core_axis_name="c", subcore_axis_name="s")
    return pl.kernel(
        _sc_body,
        out_type=(jax.ShapeDtypeStruct((2, NP, 128), jnp.float32),
                  jax.ShapeDtypeStruct((2 * NP,), jnp.float32)),
        mesh=mesh,
        compiler_params=pltpu.CompilerParams(needs_layout_passes=False),
        scratch_types=[
            pltpu.VMEM((16,), jnp.float32),
            pltpu.VMEM((R, CHUNK), jnp.int32),
            pltpu.VMEM((R, CHUNK), jnp.int32),
            pltpu.VMEM((R, CHUNK), jnp.int32),
            pltpu.VMEM((R, CHUNK), jnp.float32),
            pltpu.VMEM((R, CHUNK), jnp.float32),
            pltpu.VMEM((R, CHUNK), jnp.float32),
            pltpu.VMEM((R, CHUNK, 128), jnp.float32),
            pltpu.VMEM_SHARED((NP, 128), jnp.float32),
            pltpu.VMEM_SHARED((NP,), jnp.float32),
            pltpu.VMEM_SHARED((NP,), jnp.float32),
            pltpu.VMEM_SHARED((NP,), jnp.float32),
            pltpu.SemaphoreType.DMA((R,)),
            pltpu.SemaphoreType.DMA((R,)),
            pltpu.SemaphoreType.DMA((R,)),
            pltpu.SemaphoreType.DMA((R,)),
            pltpu.SemaphoreType.DMA((R,)),
            pltpu.SemaphoreType.DMA((R,)),
            pltpu.SemaphoreType.DMA((R,)),
        ],
    )


def _tail_body(acc, den, b1_ref, b2_ref, cb1_ref, cb2_ref, cell_ref, *rest):
    prefs = rest[:-1]
    out_ref = rest[-1]
    names = _TAIL_PARAM_NAMES
    p = {n: r[...] for n, r in zip(names, prefs)}
    row = lax.broadcasted_iota(jnp.int32, (NP, 1), 0)

    def drug_tail(d, blast, batch, tag):
        g = acc[d] / (den[d] + 1e-16) + blast[...]
        x = jnp.where(row < N, _lrelu(g), 0.0)
        z = x @ p[f"{tag}att_w"] + p[f"{tag}att_b"]
        z = jnp.where(row < N, z, -1e30)
        z = z - jnp.max(z)
        ez = jnp.exp(z)
        s = ez / jnp.sum(ez)
        gid = lax.broadcasted_iota(jnp.int32, (1, NUM_G), 1)
        oh = (batch[...] == gid).astype(jnp.float32)
        gp = lax.dot_general(oh, s * x, (((0,), (0,)), ((), ())),
                             preferred_element_type=jnp.float32)
        h = _lrelu((gp @ p[f"{tag}fc1_w"] + p[f"{tag}fc1_b"]) * p[f"{tag}bn1_g"]
                   + p[f"{tag}bn1_b"])
        return h @ p[f"{tag}fc2_w"] + p[f"{tag}fc2_b"]

    h1 = drug_tail(0, b1_ref, cb1_ref, "d1.")
    h2 = drug_tail(1, b2_ref, cb2_ref, "d2.")

    cell = cell_ref[...]
    v = cell / (jnp.sqrt(jnp.sum(cell * cell, axis=1, keepdims=True)) + 1e-12)
    h = _lrelu((v @ p["c.w1"] + p["c.b1"]) * p["c.g1"] + p["c.be1"])
    h = _lrelu((h @ p["c.w2"] + p["c.b2"]) * p["c.g2"] + p["c.be2"])
    c = h @ p["c.w3"] + p["c.b3"]

    xc = jnp.concatenate([h1, h2, c], axis=1)
    for i in range(2):
        gate = jax.nn.sigmoid(xc @ p[f"m.gw{i}"] + p[f"m.gb{i}"])
        nl = _lrelu(xc @ p[f"m.nw{i}"] + p[f"m.nb{i}"])
        lin = xc @ p[f"m.lw{i}"] + p[f"m.lb{i}"]
        xc = gate * nl + (1.0 - gate) * lin
    h = _lrelu(xc @ p["s.w1"] + p["s.b1"])
    h = _lrelu(h @ p["s.w2"] + p["s.b2"])
    out_ref[...] = h @ p["s.w3"] + p["s.b3"]


_TAIL_PARAM_NAMES = None


def _edge_arrays(edge_index, extra=0):
    loop = jnp.arange(N, dtype=jnp.int32)
    padn = E_PAD - E_TOT + extra
    src = jnp.concatenate([edge_index[0], loop,
                           jnp.zeros((padn,), jnp.int32)])
    dst = jnp.concatenate([edge_index[1], loop,
                           jnp.full((padn,), GARBAGE, jnp.int32)])
    return src, dst


@jax.jit
def kernel(x1, edge_index1, batch1, x2, edge_index2, batch2, cell, params):
    global _TAIL_PARAM_NAMES
    d1, d2 = params["d1"], params["d2"]

    src1, dst1 = _edge_arrays(edge_index1)
    src2, dst2 = _edge_arrays(edge_index2, extra=CHUNK)
    src_f = jnp.concatenate([src1, src2])
    dst_f = jnp.concatenate([dst1, dst2])

    h, es, ed, m = pl.pallas_call(
        _prep0_body,
        out_shape=(jax.ShapeDtypeStruct((2 * NP, 128), jnp.float32),
                   jax.ShapeDtypeStruct((2 * NP, 1), jnp.float32),
                   jax.ShapeDtypeStruct((2 * NP, 1), jnp.float32),
                   jax.ShapeDtypeStruct((2, 16), jnp.float32)),
    )(x1, x2,
      d1["W0"], d1["as0"].reshape(-1, 1), d1["ad0"].reshape(-1, 1),
      d2["W0"], d2["as0"].reshape(-1, 1), d2["ad0"].reshape(-1, 1))

    for i in range(3):
        acc, den = _sc_layer()(
            src_f, dst_f, es.reshape(-1), ed.reshape(-1), m.reshape(-1), h)
        den = den.reshape(2, NP, 1)
        if i < 2:
            h, es, ed, m = pl.pallas_call(
                _prepmid_body,
                out_shape=(jax.ShapeDtypeStruct((2 * NP, 128), jnp.float32),
                           jax.ShapeDtypeStruct((2 * NP, 1), jnp.float32),
                           jax.ShapeDtypeStruct((2 * NP, 1), jnp.float32),
                           jax.ShapeDtypeStruct((2, 16), jnp.float32)),
            )(acc, den,
              d1[f"b{i}"].reshape(1, -1), d1[f"W{i+1}"],
              d1[f"as{i+1}"].reshape(-1, 1), d1[f"ad{i+1}"].reshape(-1, 1),
              d2[f"b{i}"].reshape(1, -1), d2[f"W{i+1}"],
              d2[f"as{i+1}"].reshape(-1, 1), d2[f"ad{i+1}"].reshape(-1, 1))

    tail = {}
    for tag, d in (("d1.", d1), ("d2.", d2)):
        for k in ("att_w", "att_b", "fc1_w", "fc1_b", "bn1_g", "bn1_b",
                  "fc2_w", "fc2_b"):
            tail[tag + k] = d[k]
    for k, v in params["cell"].items():
        tail["c." + k] = v
    for k, v in params["mfic"].items():
        tail["m." + k] = v
    for k, v in params["syn"].items():
        tail["s." + k] = v
    _TAIL_PARAM_NAMES = tuple(tail.keys())
    pvals = [tail[n] for n in _TAIL_PARAM_NAMES]

    bp1 = jnp.pad(batch1, (0, NP - N)).reshape(NP, 1)
    bp2 = jnp.pad(batch2, (0, NP - N)).reshape(NP, 1)

    return pl.pallas_call(
        _tail_body,
        out_shape=jax.ShapeDtypeStruct((NUM_G, 2), jnp.float32),
    )(acc, den, d1["b2"], d2["b2"], bp1, bp2, cell, *pvals)
